# Initial kernel scaffold; baseline (speedup 1.0000x reference)
import jax
import jax.numpy as jnp
from jax.experimental import pallas as pl


def kernel(x, embedding):
    raise NotImplementedError("write your pallas kernel here")


# speedup vs baseline: 1.0952x; 1.0952x over previous
import functools

import jax
import jax.numpy as jnp
from jax import lax
from jax.experimental import pallas as pl
from jax.experimental.pallas import tpu as pltpu
from jax.experimental.pallas import tpu_sc as plsc

_DIM = 256
_K = 8192
_N = 16384
_BM = 512
_BN = 2048
_NCHUNK = _K // _BN


def _vq_body(x2_ref, e2_ref, x_ref, emb_ref, idx_ref, msum_ref):
    i = pl.program_id(0)
    x_blk = x_ref[...]
    x2 = x2_ref[...]
    run_min = jnp.full((_BM, 1), jnp.inf, jnp.float32)
    run_idx = jnp.zeros((_BM, 1), jnp.int32)
    for j in range(_NCHUNK):
        e_blk = emb_ref[pl.ds(j * _BN, _BN), :]
        mm = lax.dot_general(x_blk, e_blk, (((1,), (1,)), ((), ())),
                             preferred_element_type=jnp.float32)
        e2c = e2_ref[:, pl.ds(j * _BN, _BN)]
        s = (x2 + e2c) - 2.0 * mm
        m = jnp.min(s, axis=1, keepdims=True)
        il = lax.broadcasted_iota(jnp.int32, (_BM, _BN), 1)
        cand = jnp.where(s == m, il, jnp.int32(2 ** 30))
        ci = jnp.min(cand, axis=1, keepdims=True)
        better = m < run_min
        run_idx = jnp.where(better, ci + j * _BN, run_idx)
        run_min = jnp.where(better, m, run_min)
    idx_ref[...] = run_idx[:, 0]
    part = jnp.sum(run_min)

    @pl.when(i == 0)
    def _():
        msum_ref[0, 0] = part

    @pl.when(i != 0)
    def _():
        msum_ref[0, 0] = msum_ref[0, 0] + part


def _vq_argmin(x2, e2, x, embedding):
    return pl.pallas_call(
        _vq_body,
        grid=(_N // _BM,),
        in_specs=[
            pl.BlockSpec((_BM, 1), lambda i: (i, 0)),
            pl.BlockSpec((1, _K), lambda i: (0, 0)),
            pl.BlockSpec((_BM, _DIM), lambda i: (i, 0)),
            pl.BlockSpec((_K, _DIM), lambda i: (0, 0)),
        ],
        out_specs=[
            pl.BlockSpec((_BM,), lambda i: (i,)),
            pl.BlockSpec((1, 1), lambda i: (0, 0), memory_space=pltpu.SMEM),
        ],
        out_shape=[
            jax.ShapeDtypeStruct((_N,), jnp.int32),
            jax.ShapeDtypeStruct((1, 1), jnp.float32),
        ],
    )(x2, e2, x, embedding)


def _sc_gather(embedding, indices):
    info = plsc.get_sparse_core_info()
    nc, ns = info.num_cores, info.num_subcores
    nw = nc * ns
    bpw = _N // nw
    ch = 256
    nch = bpw // ch
    mesh = plsc.VectorSubcoreMesh(core_axis_name="c", subcore_axis_name="s")

    @functools.partial(
        pl.kernel,
        mesh=mesh,
        out_type=jax.ShapeDtypeStruct((_N, _DIM), jnp.float32),
        scratch_types=[
            pltpu.VMEM((bpw,), jnp.int32),
            pltpu.VMEM((ch, _DIM), jnp.float32),
            pltpu.SemaphoreType.DMA,
        ],
    )
    def gk(table_hbm, idx_hbm, out_hbm, idx_v, rows_v, sem):
        wid = lax.axis_index("s") * nc + lax.axis_index("c")
        base = wid * bpw
        pltpu.sync_copy(idx_hbm.at[pl.ds(base, bpw)], idx_v)
        for c in range(nch):
            pltpu.async_copy(
                table_hbm.at[idx_v.at[pl.ds(c * ch, ch)]], rows_v, sem
            ).wait()
            pltpu.sync_copy(rows_v, out_hbm.at[pl.ds(base + c * ch, ch)])

    return gk(embedding, indices)


def kernel(x, embedding):
    input_shape = x.shape
    flat_x = x.reshape(-1, _DIM)
    x2 = jnp.sum(flat_x ** 2, axis=1, keepdims=True)
    e2 = jnp.sum(embedding ** 2, axis=1)[None, :]
    indices, msum = _vq_argmin(x2, e2, flat_x, embedding)
    q = _sc_gather(embedding, indices)
    commitment_loss = msum[0, 0] / (_N * _DIM)
    quantized = (flat_x + (q - flat_x)).reshape(input_shape)
    return quantized, indices, commitment_loss


# speedup vs baseline: 1.2888x; 1.1768x over previous
import functools

import jax
import jax.numpy as jnp
from jax import lax
from jax.experimental import pallas as pl
from jax.experimental.pallas import tpu as pltpu
from jax.experimental.pallas import tpu_sc as plsc

_DIM = 256
_K = 8192
_N = 16384
_BM = 512
_BN = 2048
_NCHUNK = _K // _BN


def _vq_body(x2_ref, e2_ref, iota_ref, x_ref, emb2_ref, idx_ref, msum_ref):
    i = pl.program_id(0)
    x_blk = x_ref[...]
    x2 = x2_ref[...]
    run_min = jnp.full((_BM, 1), jnp.inf, jnp.float32)
    run_idx = jnp.zeros((_BM, 1), jnp.float32)
    for j in range(_NCHUNK):
        e_blk = emb2_ref[pl.ds(j * _BN, _BN), :]
        mm2 = lax.dot_general(x_blk, e_blk, (((1,), (1,)), ((), ())),
                              preferred_element_type=jnp.float32)
        e2c = e2_ref[:, pl.ds(j * _BN, _BN)]
        s = (x2 + e2c) - mm2
        m = jnp.min(s, axis=1, keepdims=True)
        cand = jnp.where(s == m, iota_ref[...], jnp.float32(1e9))
        ci = jnp.min(cand, axis=1, keepdims=True)
        better = m < run_min
        run_idx = jnp.where(better, ci + jnp.float32(j * _BN), run_idx)
        run_min = jnp.where(better, m, run_min)
    idx_ref[...] = run_idx[:, 0].astype(jnp.int32)
    part = jnp.sum(run_min)

    @pl.when(i == 0)
    def _():
        msum_ref[0, 0] = part

    @pl.when(i != 0)
    def _():
        msum_ref[0, 0] = msum_ref[0, 0] + part


def _vq_argmin(x2, e2, iota_f, x, embedding_dbl):
    return pl.pallas_call(
        _vq_body,
        grid=(_N // _BM,),
        in_specs=[
            pl.BlockSpec((_BM, 1), lambda i: (i, 0)),
            pl.BlockSpec((1, _K), lambda i: (0, 0)),
            pl.BlockSpec((1, _BN), lambda i: (0, 0)),
            pl.BlockSpec((_BM, _DIM), lambda i: (i, 0)),
            pl.BlockSpec((_K, _DIM), lambda i: (0, 0)),
        ],
        out_specs=[
            pl.BlockSpec((_BM,), lambda i: (i,)),
            pl.BlockSpec((1, 1), lambda i: (0, 0), memory_space=pltpu.SMEM),
        ],
        out_shape=[
            jax.ShapeDtypeStruct((_N,), jnp.int32),
            jax.ShapeDtypeStruct((1, 1), jnp.float32),
        ],
    )(x2, e2, iota_f, x, embedding_dbl)


def _sc_gather(embedding, indices):
    info = plsc.get_sparse_core_info()
    nc, ns = info.num_cores, info.num_subcores
    nw = nc * ns
    bpw = _N // nw
    ch = 256
    nch = bpw // ch
    mesh = plsc.VectorSubcoreMesh(core_axis_name="c", subcore_axis_name="s")

    @functools.partial(
        pl.kernel,
        mesh=mesh,
        out_type=jax.ShapeDtypeStruct((_N, _DIM), jnp.float32),
        scratch_types=[
            pltpu.VMEM((bpw,), jnp.int32),
            pltpu.VMEM((ch, _DIM), jnp.float32),
            pltpu.SemaphoreType.DMA,
        ],
    )
    def gk(table_hbm, idx_hbm, out_hbm, idx_v, rows_v, sem):
        wid = lax.axis_index("s") * nc + lax.axis_index("c")
        base = wid * bpw
        pltpu.sync_copy(idx_hbm.at[pl.ds(base, bpw)], idx_v)
        for c in range(nch):
            pltpu.async_copy(
                table_hbm.at[idx_v.at[pl.ds(c * ch, ch)]], rows_v, sem
            ).wait()
            pltpu.sync_copy(rows_v, out_hbm.at[pl.ds(base + c * ch, ch)])

    return gk(embedding, indices)


def kernel(x, embedding):
    input_shape = x.shape
    flat_x = x.reshape(-1, _DIM)
    x2 = jnp.sum(flat_x ** 2, axis=1, keepdims=True)
    e2 = jnp.sum(embedding ** 2, axis=1)[None, :]
    iota_f = jnp.arange(_BN, dtype=jnp.float32)[None, :]
    indices, msum = _vq_argmin(x2, e2, iota_f, flat_x, embedding + embedding)
    q = _sc_gather(embedding, indices)
    commitment_loss = msum[0, 0] / (_N * _DIM)
    quantized = q.reshape(input_shape)
    return quantized, indices, commitment_loss


# speedup vs baseline: 1.5598x; 1.2103x over previous
import functools

import jax
import jax.numpy as jnp
from jax import lax
from jax.experimental import pallas as pl
from jax.experimental.pallas import tpu as pltpu
from jax.experimental.pallas import tpu_sc as plsc

_DIM = 256
_K = 8192
_N = 16384
_BM = 512
_BN = 256
_NCHUNK = _K // _BN


def _vq_body(x2_ref, e2_ref, iota_ref, x_ref, emb_ref, idx_ref, msum_ref,
             emb2_ref):
    i = pl.program_id(0)

    @pl.when(i == 0)
    def _():
        emb2_ref[...] = emb_ref[...] + emb_ref[...]

    x_blk = x_ref[...]
    x2 = x2_ref[...]
    iof = iota_ref[...]
    accv = None
    acci = None
    for j in range(_NCHUNK):
        e_blk = emb2_ref[pl.ds(j * _BN, _BN), :]
        mm2 = lax.dot_general(x_blk, e_blk, (((1,), (1,)), ((), ())),
                              preferred_element_type=jnp.float32)
        e2c = e2_ref[:, pl.ds(j * _BN, _BN)]
        s = (x2 + e2c) - mm2
        if accv is None:
            accv = s
            acci = jnp.broadcast_to(iof, (_BM, _BN))
        else:
            take = s < accv
            accv = jnp.where(take, s, accv)
            acci = jnp.where(take, iof + jnp.float32(j * _BN), acci)
    m = jnp.min(accv, axis=1, keepdims=True)
    cand = jnp.where(accv == m, acci, jnp.float32(1e9))
    ci = jnp.min(cand, axis=1, keepdims=True)
    idx_ref[...] = ci[:, 0].astype(jnp.int32)
    part = jnp.sum(m)

    @pl.when(i == 0)
    def _():
        msum_ref[0, 0] = part

    @pl.when(i != 0)
    def _():
        msum_ref[0, 0] = msum_ref[0, 0] + part


def _vq_argmin(x2, e2, iota_f, x, embedding):
    return pl.pallas_call(
        _vq_body,
        grid=(_N // _BM,),
        in_specs=[
            pl.BlockSpec((_BM, 1), lambda i: (i, 0)),
            pl.BlockSpec((1, _K), lambda i: (0, 0)),
            pl.BlockSpec((1, _BN), lambda i: (0, 0)),
            pl.BlockSpec((_BM, _DIM), lambda i: (i, 0)),
            pl.BlockSpec((_K, _DIM), lambda i: (0, 0)),
        ],
        out_specs=[
            pl.BlockSpec((_BM,), lambda i: (i,)),
            pl.BlockSpec((1, 1), lambda i: (0, 0), memory_space=pltpu.SMEM),
        ],
        out_shape=[
            jax.ShapeDtypeStruct((_N,), jnp.int32),
            jax.ShapeDtypeStruct((1, 1), jnp.float32),
        ],
        scratch_shapes=[pltpu.VMEM((_K, _DIM), jnp.float32)],
    )(x2, e2, iota_f, x, embedding)


def _sc_gather(embedding, indices):
    info = plsc.get_sparse_core_info()
    nc, ns = info.num_cores, info.num_subcores
    nw = nc * ns
    bpw = _N // nw
    ch = 256
    nch = bpw // ch
    mesh = plsc.VectorSubcoreMesh(core_axis_name="c", subcore_axis_name="s")

    @functools.partial(
        pl.kernel,
        mesh=mesh,
        out_type=jax.ShapeDtypeStruct((_N, _DIM), jnp.float32),
        scratch_types=[
            pltpu.VMEM((bpw,), jnp.int32),
            pltpu.VMEM((ch, _DIM), jnp.float32),
            pltpu.SemaphoreType.DMA,
        ],
    )
    def gk(table_hbm, idx_hbm, out_hbm, idx_v, rows_v, sem):
        wid = lax.axis_index("s") * nc + lax.axis_index("c")
        base = wid * bpw
        pltpu.sync_copy(idx_hbm.at[pl.ds(base, bpw)], idx_v)
        for c in range(nch):
            pltpu.async_copy(
                table_hbm.at[idx_v.at[pl.ds(c * ch, ch)]], rows_v, sem
            ).wait()
            pltpu.sync_copy(rows_v, out_hbm.at[pl.ds(base + c * ch, ch)])

    return gk(embedding, indices)


def kernel(x, embedding):
    input_shape = x.shape
    flat_x = x.reshape(-1, _DIM)
    x2 = jnp.sum(flat_x ** 2, axis=1, keepdims=True)
    e2 = jnp.sum(embedding ** 2, axis=1)[None, :]
    iota_f = jnp.arange(_BN, dtype=jnp.float32)[None, :]
    indices, msum = _vq_argmin(x2, e2, iota_f, flat_x, embedding)
    q = _sc_gather(embedding, indices)
    commitment_loss = msum[0, 0] / (_N * _DIM)
    quantized = q.reshape(input_shape)
    return quantized, indices, commitment_loss


# speedup vs baseline: 1.6665x; 1.0684x over previous
import functools

import jax
import jax.numpy as jnp
from jax import lax
from jax.experimental import pallas as pl
from jax.experimental.pallas import tpu as pltpu
from jax.experimental.pallas import tpu_sc as plsc

_DIM = 256
_K = 8192
_N = 16384
_BM = 512
_BN = 256
_NCHUNK = _K // _BN


def _vq_body(iota_ref, x_ref, emb_ref, idx_ref, msum_ref, emb2_ref, e2_ref):
    i = pl.program_id(0)

    @pl.when(i == 0)
    def _():
        emb2_ref[...] = emb_ref[...] + emb_ref[...]
        sq = emb_ref[...] * emb_ref[...]
        e2_ref[...] = lax.dot_general(
            jnp.ones((1, _DIM), jnp.float32), sq, (((1,), (1,)), ((), ())),
            preferred_element_type=jnp.float32)

    x_blk = x_ref[...]
    x2 = jnp.sum(x_blk * x_blk, axis=1, keepdims=True)
    iof = iota_ref[...]
    accv = None
    acci = None
    for j in range(_NCHUNK):
        e_blk = emb2_ref[pl.ds(j * _BN, _BN), :]
        mm2 = lax.dot_general(x_blk, e_blk, (((1,), (1,)), ((), ())),
                              preferred_element_type=jnp.float32)
        e2c = e2_ref[:, pl.ds(j * _BN, _BN)]
        s = (x2 + e2c) - mm2
        if accv is None:
            accv = s
            acci = jnp.broadcast_to(iof, (_BM, _BN))
        else:
            take = s < accv
            accv = jnp.where(take, s, accv)
            acci = jnp.where(take, iof + jnp.float32(j * _BN), acci)
    m = jnp.min(accv, axis=1, keepdims=True)
    cand = jnp.where(accv == m, acci, jnp.float32(1e9))
    ci = jnp.min(cand, axis=1, keepdims=True)
    idx_ref[...] = ci[:, 0].astype(jnp.int32)
    part = jnp.sum(m)

    @pl.when(i == 0)
    def _():
        msum_ref[0, 0] = part

    @pl.when(i != 0)
    def _():
        msum_ref[0, 0] = msum_ref[0, 0] + part


def _vq_argmin(iota_f, x, embedding):
    return pl.pallas_call(
        _vq_body,
        grid=(_N // _BM,),
        in_specs=[
            pl.BlockSpec((1, _BN), lambda i: (0, 0)),
            pl.BlockSpec((_BM, _DIM), lambda i: (i, 0)),
            pl.BlockSpec((_K, _DIM), lambda i: (0, 0)),
        ],
        out_specs=[
            pl.BlockSpec((_BM,), lambda i: (i,)),
            pl.BlockSpec((1, 1), lambda i: (0, 0), memory_space=pltpu.SMEM),
        ],
        out_shape=[
            jax.ShapeDtypeStruct((_N,), jnp.int32),
            jax.ShapeDtypeStruct((1, 1), jnp.float32),
        ],
        scratch_shapes=[
            pltpu.VMEM((_K, _DIM), jnp.float32),
            pltpu.VMEM((1, _K), jnp.float32),
        ],
    )(iota_f, x, embedding)


def _sc_gather(embedding, indices):
    info = plsc.get_sparse_core_info()
    nc, ns = info.num_cores, info.num_subcores
    nw = nc * ns
    bpw = _N // nw
    ch = 128
    nch = bpw // ch
    mesh = plsc.VectorSubcoreMesh(core_axis_name="c", subcore_axis_name="s")

    @functools.partial(
        pl.kernel,
        mesh=mesh,
        out_type=jax.ShapeDtypeStruct((_N, _DIM), jnp.float32),
        scratch_types=[
            pltpu.VMEM((bpw,), jnp.int32),
            pltpu.VMEM((ch, _DIM), jnp.float32),
            pltpu.VMEM((ch, _DIM), jnp.float32),
            pltpu.SemaphoreType.DMA,
            pltpu.SemaphoreType.DMA,
        ],
    )
    def gk(table_hbm, idx_hbm, out_hbm, idx_v, rows0, rows1, sem0, sem1):
        wid = lax.axis_index("s") * nc + lax.axis_index("c")
        base = wid * bpw
        pltpu.sync_copy(idx_hbm.at[pl.ds(base, bpw)], idx_v)
        bufs = (rows0, rows1)
        sems = (sem0, sem1)
        prev = pltpu.async_copy(table_hbm.at[idx_v.at[pl.ds(0, ch)]],
                                bufs[0], sems[0])
        for c in range(1, nch):
            nxt = pltpu.async_copy(table_hbm.at[idx_v.at[pl.ds(c * ch, ch)]],
                                   bufs[c % 2], sems[c % 2])
            prev.wait()
            pltpu.sync_copy(bufs[(c - 1) % 2],
                            out_hbm.at[pl.ds(base + (c - 1) * ch, ch)])
            prev = nxt
        prev.wait()
        pltpu.sync_copy(bufs[(nch - 1) % 2],
                        out_hbm.at[pl.ds(base + (nch - 1) * ch, ch)])

    return gk(embedding, indices)


def kernel(x, embedding):
    input_shape = x.shape
    flat_x = x.reshape(-1, _DIM)
    iota_f = jnp.arange(_BN, dtype=jnp.float32)[None, :]
    indices, msum = _vq_argmin(iota_f, flat_x, embedding)
    q = _sc_gather(embedding, indices)
    commitment_loss = msum[0, 0] / (_N * _DIM)
    quantized = q.reshape(input_shape)
    return quantized, indices, commitment_loss
